# baseline (speedup 1.0000x reference)
import jax
import jax.numpy as jnp
from jax import lax
from jax.experimental import pallas as pl
from jax.experimental.pallas import tpu as pltpu
from jax.experimental.pallas import tpu_sc as plsc

N = 4
M = 16384
MP = 65536
NNZ = 262144
F = 64
F2 = 2 * F
NC = 2
NS = 16
L = 16
RB = 4096
NBLK = MP // RB
GPT = 8
HG = GPT // 2
CHUNK = GPT * L
HROW = HG * L
ZR = RB // NS
PKR = 64
RPT = M // NS
FQ = F // L
BPAD = 64


def _body(xin, xi, bp, zin, y, xp,
          shared, bp_v, xi_v, xg_v, sc_v, lidx_v, ca_v,
          po_v, pa_v, pb_v, st_v,
          idx_sem, xg_sem, sc_sem):
    c = lax.axis_index("c")
    s = lax.axis_index("s")
    iota = lax.iota(jnp.int32, L)
    zv = jnp.zeros((L,), jnp.int32)
    pltpu.sync_copy(bp, bp_v)
    pbase = c * M

    def pack(w, carry):
        m0 = s * RPT + w * PKR
        pltpu.sync_copy(xin.at[2 * c, pl.ds(m0, PKR), :], pa_v)
        pltpu.sync_copy(xin.at[2 * c + 1, pl.ds(m0, PKR), :], pb_v)
        for r in range(PKR):
            for q in range(FQ):
                po_v[r, pl.ds(q * L, L)] = pa_v[r, pl.ds(q * L, L)]
                po_v[r, pl.ds(F + q * L, L)] = pb_v[r, pl.ds(q * L, L)]
        pltpu.sync_copy(po_v, xp.at[pl.ds(pbase + m0, PKR)])
        return carry

    lax.fori_loop(0, RPT // PKR, pack, 0)
    plsc.subcore_barrier()

    def fire_idx(gb, t, buf):
        psc = jnp.minimum((gb + t * GPT) * L, NNZ - CHUNK)
        pltpu.async_copy(xi.at[pl.ds(3 * psc, 3 * CHUNK)],
                         xi_v.at[pl.ds(buf * 3 * CHUNK, 3 * CHUNK)], idx_sem)

    def drain_idx():
        pltpu.make_async_copy(xi.at[pl.ds(0, 3 * CHUNK)],
                              xi_v.at[pl.ds(0, 3 * CHUNK)], idx_sem).wait()

    def fire_gathers(buf):
        for u in range(CHUNK // L):
            colu = plsc.load_gather(
                xi_v, [buf * 3 * CHUNK + 3 * (u * L + iota)])
            plsc.store_scatter(ca_v, [buf * CHUNK + u * L + iota],
                               colu + pbase)
        pltpu.async_copy(
            xp.at[ca_v.at[pl.ds(buf * CHUNK, CHUNK)]],
            xg_v.at[pl.ds(buf * CHUNK, CHUNK)], xg_sem)

    def drain_gathers():
        pltpu.make_async_copy(xp.at[pl.ds(0, CHUNK)],
                              xg_v.at[pl.ds(0, CHUNK)], xg_sem).wait()

    def drain_scatters():
        pltpu.make_async_copy(sc_v, shared.at[pl.ds(0, CHUNK)],
                              sc_sem).wait()

    def pair_body(b, pcarry):
        start = plsc.load_gather(bp_v, [zv + b])[0]
        end = plsc.load_gather(bp_v, [zv + (b + 1)])[0]
        g0 = start // L
        g1 = (end + (L - 1)) // L
        mt = (g1 - g0 + (NS - 1)) // NS
        gb = g0 + s * mt
        ge = jnp.minimum(gb + mt, g1)
        rowoff = b * RB
        start_v = zv + start
        end_v = zv + end
        hi_v = jnp.minimum(end_v, zv + ge * L)

        pltpu.sync_copy(zin, shared.at[pl.ds(s * ZR, ZR)])
        plsc.subcore_barrier()

        p0 = jnp.minimum(gb * L, NNZ - CHUNK)
        pltpu.sync_copy(xi.at[pl.ds(3 * p0, 3 * CHUNK)],
                        xi_v.at[pl.ds(0, 3 * CHUNK)])
        fire_gathers(0)
        fire_idx(gb, 1, 1)

        def trip(t):
            buf = t % 2
            nbuf = 1 - buf
            psc = jnp.minimum((gb + t * GPT) * L, NNZ - CHUNK)
            lo_v = jnp.maximum(start_v, zv + (gb + t * GPT) * L)

            drain_gathers()
            drain_idx()

            @pl.when(t > 0)
            def _():
                drain_scatters()

            fire_gathers(nbuf)

            @plsc.parallel_loop(0, GPT, 1, unroll=2, carry=jnp.int32(0))
            def grp(j, carry):
                co = buf * CHUNK + j * L
                xo = buf * 3 * CHUNK + 3 * j * L
                pos = zv + psc + j * L + iota
                msk = (pos >= lo_v) & (pos < hi_v)
                rowj = plsc.load_gather(xi_v, [xo + 3 * iota + 1])
                lidx = jnp.where(msk, rowj - rowoff, RB + s)
                plsc.store_scatter(lidx_v, [zv + buf, j * L + iota], lidx)
                for k in range(L):
                    kv = zv + (co + k)
                    cv = zv + (j * L + k)
                    vb = plsc.bitcast(
                        plsc.load_gather(xi_v, [zv + (xo + 3 * k + 2)]),
                        jnp.float32)
                    for q in range(F2 // L):
                        cix = q * L + iota
                        xrow = plsc.load_gather(xg_v, [kv, cix])
                        plsc.store_scatter(sc_v, [cv, cix], vb * xrow)
                return carry

            pltpu.async_copy(sc_v, shared.at[lidx_v.at[buf]],
                             sc_sem, add=True)
            fire_idx(gb, t + 2, buf)
            return t + 1

        tT = lax.while_loop(lambda t: gb + t * GPT < ge, trip, jnp.int32(0))
        drain_gathers()
        drain_idx()

        @pl.when(tT > 0)
        def _():
            drain_scatters()
        plsc.subcore_barrier()

        def unpk(w, carry):
            r0 = s * ZR + w * PKR
            pltpu.sync_copy(shared.at[pl.ds(r0, PKR)], st_v)
            for r in range(PKR):
                for q in range(FQ):
                    pa_v[r, pl.ds(q * L, L)] = st_v[r, pl.ds(q * L, L)]
                    pb_v[r, pl.ds(q * L, L)] = st_v[r, pl.ds(F + q * L, L)]
            ob = rowoff + r0
            pltpu.sync_copy(pa_v, y.at[2 * c, pl.ds(ob, PKR), :])
            pltpu.sync_copy(pb_v, y.at[2 * c + 1, pl.ds(ob, PKR), :])
            return carry

        lax.fori_loop(0, ZR // PKR, unpk, 0)
        plsc.subcore_barrier()
        return pcarry

    lax.fori_loop(0, NBLK, pair_body, 0)


_spmm = pl.kernel(
    _body,
    out_type=(
        jax.ShapeDtypeStruct((N, MP, F), jnp.float32),
        jax.ShapeDtypeStruct((NC * M, F2), jnp.float32),
    ),
    mesh=plsc.VectorSubcoreMesh(core_axis_name="c", subcore_axis_name="s"),
    compiler_params=pltpu.CompilerParams(needs_layout_passes=False),
    scratch_types=[
        pltpu.VMEM_SHARED((RB + NS, F2), jnp.float32),
        pltpu.VMEM((BPAD,), jnp.int32),
        pltpu.VMEM((2 * 3 * CHUNK,), jnp.int32),
        pltpu.VMEM((2 * CHUNK, F2), jnp.float32),
        pltpu.VMEM((CHUNK, F2), jnp.float32),
        pltpu.VMEM((2, CHUNK), jnp.int32),
        pltpu.VMEM((2 * CHUNK,), jnp.int32),
        pltpu.VMEM((PKR, F2), jnp.float32),
        pltpu.VMEM((PKR, F), jnp.float32),
        pltpu.VMEM((PKR, F), jnp.float32),
        pltpu.VMEM((PKR, F2), jnp.float32),
        pltpu.SemaphoreType.DMA,
        pltpu.SemaphoreType.DMA,
        pltpu.SemaphoreType.DMA,
    ],
)


@jax.jit
def kernel(inputs, mat_rows, mat_cols, mat_vals):
    bounds = jnp.arange(0, MP + 1, RB, dtype=jnp.int32)
    bp1 = jnp.searchsorted(mat_rows, bounds, side="left").astype(jnp.int32)
    bp = jnp.concatenate([bp1, jnp.zeros((BPAD - NBLK - 1,), jnp.int32)])
    vi = jax.lax.bitcast_convert_type(mat_vals, jnp.int32)
    xi = jnp.stack([mat_cols, mat_rows, vi], axis=1).reshape(-1)
    zin = jnp.zeros((ZR, F2), jnp.float32)
    y, _ = _spmm(inputs, xi, bp, zin)
    return y


# speedup vs baseline: 1.0405x; 1.0405x over previous
import jax
import jax.numpy as jnp
from jax import lax
from jax.experimental import pallas as pl
from jax.experimental.pallas import tpu as pltpu
from jax.experimental.pallas import tpu_sc as plsc

N = 4
M = 16384
MP = 65536
NNZ = 262144
F = 64
F2 = 2 * F
NC = 2
NS = 16
L = 16
RB = 4096
NBLK = MP // RB
GPT = 8
HG = GPT // 2
CHUNK = GPT * L
HROW = HG * L
ZR = RB // NS
PKR = 64
RPT = M // NS
FQ = F // L
BPAD = 64


def _body(xin, xi, bp, y, xp,
          shared, bp_v, xi_v, xg_v, sc_v, lidx_v, ca_v,
          po_v, pa_v, pb_v, st_v,
          idx_sem, xg_sem, sc_sem):
    c = lax.axis_index("c")
    s = lax.axis_index("s")
    iota = lax.iota(jnp.int32, L)
    zv = jnp.zeros((L,), jnp.int32)
    pltpu.sync_copy(bp, bp_v)
    pbase = c * M

    def pack(w, carry):
        m0 = s * RPT + w * PKR
        pltpu.sync_copy(xin.at[2 * c, pl.ds(m0, PKR), :], pa_v)
        pltpu.sync_copy(xin.at[2 * c + 1, pl.ds(m0, PKR), :], pb_v)
        for r in range(PKR):
            for q in range(FQ):
                po_v[r, pl.ds(q * L, L)] = pa_v[r, pl.ds(q * L, L)]
                po_v[r, pl.ds(F + q * L, L)] = pb_v[r, pl.ds(q * L, L)]
        pltpu.sync_copy(po_v, xp.at[pl.ds(pbase + m0, PKR)])
        return carry

    lax.fori_loop(0, RPT // PKR, pack, 0)
    for r in range(PKR):
        for q in range(F2 // L):
            po_v[r, pl.ds(q * L, L)] = jnp.zeros((L,), jnp.float32)
    plsc.subcore_barrier()

    def fire_idx(gb, t, buf):
        psc = jnp.minimum((gb + t * GPT) * L, NNZ - CHUNK)
        pltpu.async_copy(xi.at[pl.ds(3 * psc, 3 * CHUNK)],
                         xi_v.at[pl.ds(buf * 3 * CHUNK, 3 * CHUNK)], idx_sem)

    def drain_idx():
        pltpu.make_async_copy(xi.at[pl.ds(0, 3 * CHUNK)],
                              xi_v.at[pl.ds(0, 3 * CHUNK)], idx_sem).wait()

    def fire_gathers(buf):
        for u in range(CHUNK // L):
            colu = plsc.load_gather(
                xi_v, [buf * 3 * CHUNK + 3 * (u * L + iota)])
            plsc.store_scatter(ca_v, [buf * CHUNK + u * L + iota],
                               colu + pbase)
        pltpu.async_copy(
            xp.at[ca_v.at[pl.ds(buf * CHUNK, CHUNK)]],
            xg_v.at[pl.ds(buf * CHUNK, CHUNK)], xg_sem)

    def drain_gathers():
        pltpu.make_async_copy(xp.at[pl.ds(0, CHUNK)],
                              xg_v.at[pl.ds(0, CHUNK)], xg_sem).wait()

    def drain_scatters():
        pltpu.make_async_copy(sc_v, shared.at[pl.ds(0, CHUNK)],
                              sc_sem).wait()

    def pair_body(b, pcarry):
        start = plsc.load_gather(bp_v, [zv + b])[0]
        end = plsc.load_gather(bp_v, [zv + (b + 1)])[0]
        g0 = start // L
        g1 = (end + (L - 1)) // L
        mt = (g1 - g0 + (NS - 1)) // NS
        gb = g0 + s * mt
        ge = jnp.minimum(gb + mt, g1)
        rowoff = b * RB
        start_v = zv + start
        end_v = zv + end
        hi_v = jnp.minimum(end_v, zv + ge * L)

        for z in range(ZR // PKR):
            pltpu.sync_copy(po_v, shared.at[pl.ds(s * ZR + z * PKR, PKR)])
        plsc.subcore_barrier()

        p0 = jnp.minimum(gb * L, NNZ - CHUNK)
        pltpu.sync_copy(xi.at[pl.ds(3 * p0, 3 * CHUNK)],
                        xi_v.at[pl.ds(0, 3 * CHUNK)])
        fire_gathers(0)
        fire_idx(gb, 1, 1)

        def trip(t):
            buf = t % 2
            nbuf = 1 - buf
            psc = jnp.minimum((gb + t * GPT) * L, NNZ - CHUNK)
            lo_v = jnp.maximum(start_v, zv + (gb + t * GPT) * L)

            drain_gathers()
            drain_idx()

            @pl.when(t > 0)
            def _():
                drain_scatters()

            fire_gathers(nbuf)

            @plsc.parallel_loop(0, GPT, 1, unroll=2, carry=jnp.int32(0))
            def grp(j, carry):
                co = buf * CHUNK + j * L
                xo = buf * 3 * CHUNK + 3 * j * L
                pos = zv + psc + j * L + iota
                msk = (pos >= lo_v) & (pos < hi_v)
                rowj = plsc.load_gather(xi_v, [xo + 3 * iota + 1])
                lidx = jnp.where(msk, rowj - rowoff, RB + s)
                plsc.store_scatter(lidx_v, [zv + buf, j * L + iota], lidx)
                for k in range(L):
                    kv = zv + (co + k)
                    cv = zv + (j * L + k)
                    vb = plsc.bitcast(
                        plsc.load_gather(xi_v, [zv + (xo + 3 * k + 2)]),
                        jnp.float32)
                    for q in range(F2 // L):
                        cix = q * L + iota
                        xrow = plsc.load_gather(xg_v, [kv, cix])
                        plsc.store_scatter(sc_v, [cv, cix], vb * xrow)
                return carry

            pltpu.async_copy(sc_v, shared.at[lidx_v.at[buf]],
                             sc_sem, add=True)
            fire_idx(gb, t + 2, buf)
            return t + 1

        tT = lax.while_loop(lambda t: gb + t * GPT < ge, trip, jnp.int32(0))
        drain_gathers()
        drain_idx()

        @pl.when(tT > 0)
        def _():
            drain_scatters()
        plsc.subcore_barrier()

        def unpk(w, carry):
            r0 = s * ZR + w * PKR
            pltpu.sync_copy(shared.at[pl.ds(r0, PKR)], st_v)
            for r in range(PKR):
                for q in range(FQ):
                    pa_v[r, pl.ds(q * L, L)] = st_v[r, pl.ds(q * L, L)]
                    pb_v[r, pl.ds(q * L, L)] = st_v[r, pl.ds(F + q * L, L)]
            ob = rowoff + r0
            pltpu.sync_copy(pa_v, y.at[2 * c, pl.ds(ob, PKR), :])
            pltpu.sync_copy(pb_v, y.at[2 * c + 1, pl.ds(ob, PKR), :])
            return carry

        lax.fori_loop(0, ZR // PKR, unpk, 0)
        plsc.subcore_barrier()
        return pcarry

    lax.fori_loop(0, NBLK, pair_body, 0)


_spmm = pl.kernel(
    _body,
    out_type=(
        jax.ShapeDtypeStruct((N, MP, F), jnp.float32),
        jax.ShapeDtypeStruct((NC * M, F2), jnp.float32),
    ),
    mesh=plsc.VectorSubcoreMesh(core_axis_name="c", subcore_axis_name="s"),
    compiler_params=pltpu.CompilerParams(needs_layout_passes=False),
    scratch_types=[
        pltpu.VMEM_SHARED((RB + NS, F2), jnp.float32),
        pltpu.VMEM((BPAD,), jnp.int32),
        pltpu.VMEM((2 * 3 * CHUNK,), jnp.int32),
        pltpu.VMEM((2 * CHUNK, F2), jnp.float32),
        pltpu.VMEM((CHUNK, F2), jnp.float32),
        pltpu.VMEM((2, CHUNK), jnp.int32),
        pltpu.VMEM((2 * CHUNK,), jnp.int32),
        pltpu.VMEM((PKR, F2), jnp.float32),
        pltpu.VMEM((PKR, F), jnp.float32),
        pltpu.VMEM((PKR, F), jnp.float32),
        pltpu.VMEM((PKR, F2), jnp.float32),
        pltpu.SemaphoreType.DMA,
        pltpu.SemaphoreType.DMA,
        pltpu.SemaphoreType.DMA,
    ],
)


@jax.jit
def kernel(inputs, mat_rows, mat_cols, mat_vals):
    bounds = jnp.arange(0, MP + 1, RB, dtype=jnp.int32)
    bp1 = jnp.searchsorted(mat_rows, bounds, side="left").astype(jnp.int32)
    bp = jnp.concatenate([bp1, jnp.zeros((BPAD - NBLK - 1,), jnp.int32)])
    vi = jax.lax.bitcast_convert_type(mat_vals, jnp.int32)
    xi = jnp.stack([mat_cols, mat_rows, vi], axis=1).reshape(-1)
    y, _ = _spmm(inputs, xi, bp)
    return y


# speedup vs baseline: 1.1675x; 1.1221x over previous
import jax
import jax.numpy as jnp
from jax import lax
from jax.experimental import pallas as pl
from jax.experimental.pallas import tpu as pltpu
from jax.experimental.pallas import tpu_sc as plsc

N = 4
M = 16384
MP = 65536
NNZ = 262144
F = 64
F2 = 2 * F
NC = 2
NS = 16
L = 16
RB = 4096
NBLK = MP // RB
GPT = 8
HG = GPT // 2
CHUNK = GPT * L
HROW = HG * L
ZR = RB // NS
PKR = 32
RPT = M // NS
FQ = F // L
BPAD = 64


def _body(xin, cols, vals, rows, bp, y, xp,
          shared, bp_v, cols_v, vals_v, rows_v, xg_v, sc_v, lidx_v, ca_v,
          po_v, pa_v, pb_v, st_v,
          idx_sem, xg_sem, sc_sem):
    c = lax.axis_index("c")
    s = lax.axis_index("s")
    iota = lax.iota(jnp.int32, L)
    zv = jnp.zeros((L,), jnp.int32)
    pltpu.sync_copy(bp, bp_v)
    pbase = c * M

    def pack(w, carry):
        m0 = s * RPT + w * PKR
        pltpu.sync_copy(xin.at[2 * c, pl.ds(m0, PKR), :], pa_v)
        pltpu.sync_copy(xin.at[2 * c + 1, pl.ds(m0, PKR), :], pb_v)
        for r in range(PKR):
            for q in range(FQ):
                po_v[r, pl.ds(q * L, L)] = pa_v[r, pl.ds(q * L, L)]
                po_v[r, pl.ds(F + q * L, L)] = pb_v[r, pl.ds(q * L, L)]
        pltpu.sync_copy(po_v, xp.at[pl.ds(pbase + m0, PKR)])
        return carry

    lax.fori_loop(0, RPT // PKR, pack, 0)
    for r in range(PKR):
        for q in range(F2 // L):
            po_v[r, pl.ds(q * L, L)] = jnp.zeros((L,), jnp.float32)
    plsc.subcore_barrier()

    def fire_idx(gb, t, buf):
        psc = jnp.minimum((gb + t * GPT) * L, NNZ - CHUNK)
        pltpu.async_copy(cols.at[pl.ds(psc, CHUNK)],
                         cols_v.at[pl.ds(buf * CHUNK, CHUNK)], idx_sem)
        pltpu.async_copy(vals.at[pl.ds(psc, CHUNK)],
                         vals_v.at[pl.ds(buf * CHUNK, CHUNK)], idx_sem)
        pltpu.async_copy(rows.at[pl.ds(psc, CHUNK)],
                         rows_v.at[pl.ds(buf * CHUNK, CHUNK)], idx_sem)

    def drain_idx():
        pltpu.make_async_copy(cols.at[pl.ds(0, CHUNK)],
                              cols_v.at[pl.ds(0, CHUNK)], idx_sem).wait()
        pltpu.make_async_copy(vals.at[pl.ds(0, CHUNK)],
                              vals_v.at[pl.ds(0, CHUNK)], idx_sem).wait()
        pltpu.make_async_copy(rows.at[pl.ds(0, CHUNK)],
                              rows_v.at[pl.ds(0, CHUNK)], idx_sem).wait()

    def fire_gathers(buf):
        for u in range(CHUNK // L):
            colu = plsc.load_gather(cols_v, [buf * CHUNK + u * L + iota])
            plsc.store_scatter(ca_v, [buf * CHUNK + u * L + iota],
                               colu + pbase)
        pltpu.async_copy(
            xp.at[ca_v.at[pl.ds(buf * CHUNK, CHUNK)]],
            xg_v.at[pl.ds(buf * CHUNK, CHUNK)], xg_sem)

    def drain_gathers():
        pltpu.make_async_copy(xp.at[pl.ds(0, CHUNK)],
                              xg_v.at[pl.ds(0, CHUNK)], xg_sem).wait()

    def drain_scatters():
        pltpu.make_async_copy(sc_v, shared.at[pl.ds(0, CHUNK)],
                              sc_sem).wait()

    def pair_body(b, pcarry):
        start = plsc.load_gather(bp_v, [zv + b])[0]
        end = plsc.load_gather(bp_v, [zv + (b + 1)])[0]
        g0 = start // L
        g1 = (end + (L - 1)) // L
        mt = (g1 - g0 + (NS - 1)) // NS
        gb = g0 + s * mt
        ge = jnp.minimum(gb + mt, g1)
        rowoff = b * RB
        start_v = zv + start
        end_v = zv + end
        hi_v = jnp.minimum(end_v, zv + ge * L)

        for z in range(ZR // PKR):
            pltpu.sync_copy(po_v, shared.at[pl.ds(s * ZR + z * PKR, PKR)])
        plsc.subcore_barrier()

        p0 = jnp.minimum(gb * L, NNZ - CHUNK)
        pltpu.sync_copy(cols.at[pl.ds(p0, CHUNK)], cols_v.at[pl.ds(0, CHUNK)])
        pltpu.sync_copy(vals.at[pl.ds(p0, CHUNK)], vals_v.at[pl.ds(0, CHUNK)])
        pltpu.sync_copy(rows.at[pl.ds(p0, CHUNK)], rows_v.at[pl.ds(0, CHUNK)])
        fire_gathers(0)
        fire_idx(gb, 1, 1)

        def trip(t):
            buf = t % 2
            nbuf = 1 - buf
            psc = jnp.minimum((gb + t * GPT) * L, NNZ - CHUNK)
            lo_v = jnp.maximum(start_v, zv + (gb + t * GPT) * L)

            drain_gathers()
            drain_idx()

            @pl.when(t > 0)
            def _():
                drain_scatters()

            fire_gathers(nbuf)

            @plsc.parallel_loop(0, GPT, 1, unroll=2, carry=jnp.int32(0))
            def grp(j, carry):
                co = buf * CHUNK + j * L
                pos = zv + psc + j * L + iota
                msk = (pos >= lo_v) & (pos < hi_v)
                rowj = plsc.load_gather(rows_v, [co + iota])
                lidx = jnp.where(msk, rowj - rowoff, RB + s)
                plsc.store_scatter(lidx_v, [zv + buf, j * L + iota], lidx)
                for k in range(L):
                    kv = zv + (co + k)
                    cv = zv + (j * L + k)
                    vb = plsc.load_gather(vals_v, [kv])
                    for q in range(F2 // L):
                        cix = q * L + iota
                        xrow = plsc.load_gather(xg_v, [kv, cix])
                        plsc.store_scatter(sc_v, [cv, cix], vb * xrow)
                return carry

            pltpu.async_copy(sc_v, shared.at[lidx_v.at[buf]],
                             sc_sem, add=True)
            fire_idx(gb, t + 2, buf)
            return t + 1

        tT = lax.while_loop(lambda t: gb + t * GPT < ge, trip, jnp.int32(0))
        drain_gathers()
        drain_idx()

        @pl.when(tT > 0)
        def _():
            drain_scatters()
        plsc.subcore_barrier()

        def unpk(w, carry):
            r0 = s * ZR + w * PKR
            pltpu.sync_copy(shared.at[pl.ds(r0, PKR)], st_v)
            for r in range(PKR):
                for q in range(FQ):
                    pa_v[r, pl.ds(q * L, L)] = st_v[r, pl.ds(q * L, L)]
                    pb_v[r, pl.ds(q * L, L)] = st_v[r, pl.ds(F + q * L, L)]
            ob = rowoff + r0
            pltpu.sync_copy(pa_v, y.at[2 * c, pl.ds(ob, PKR), :])
            pltpu.sync_copy(pb_v, y.at[2 * c + 1, pl.ds(ob, PKR), :])
            return carry

        lax.fori_loop(0, ZR // PKR, unpk, 0)
        plsc.subcore_barrier()
        return pcarry

    lax.fori_loop(0, NBLK, pair_body, 0)


_spmm = pl.kernel(
    _body,
    out_type=(
        jax.ShapeDtypeStruct((N, MP, F), jnp.float32),
        jax.ShapeDtypeStruct((NC * M, F2), jnp.float32),
    ),
    mesh=plsc.VectorSubcoreMesh(core_axis_name="c", subcore_axis_name="s"),
    compiler_params=pltpu.CompilerParams(needs_layout_passes=False),
    scratch_types=[
        pltpu.VMEM_SHARED((RB + NS, F2), jnp.float32),
        pltpu.VMEM((BPAD,), jnp.int32),
        pltpu.VMEM((2 * CHUNK,), jnp.int32),
        pltpu.VMEM((2 * CHUNK,), jnp.float32),
        pltpu.VMEM((2 * CHUNK,), jnp.int32),
        pltpu.VMEM((2 * CHUNK, F2), jnp.float32),
        pltpu.VMEM((CHUNK, F2), jnp.float32),
        pltpu.VMEM((2, CHUNK), jnp.int32),
        pltpu.VMEM((2 * CHUNK,), jnp.int32),
        pltpu.VMEM((PKR, F2), jnp.float32),
        pltpu.VMEM((PKR, F), jnp.float32),
        pltpu.VMEM((PKR, F), jnp.float32),
        pltpu.VMEM((PKR, F2), jnp.float32),
        pltpu.SemaphoreType.DMA,
        pltpu.SemaphoreType.DMA,
        pltpu.SemaphoreType.DMA,
    ],
)


@jax.jit
def kernel(inputs, mat_rows, mat_cols, mat_vals):
    bounds = jnp.arange(0, MP + 1, RB, dtype=jnp.int32)
    bp1 = jnp.searchsorted(mat_rows, bounds, side="left").astype(jnp.int32)
    bp = jnp.concatenate([bp1, jnp.zeros((BPAD - NBLK - 1,), jnp.int32)])
    y, _ = _spmm(inputs, mat_cols, mat_vals, mat_rows, bp)
    return y
